# Initial kernel scaffold; baseline (speedup 1.0000x reference)
import jax
import jax.numpy as jnp
from jax.experimental import pallas as pl


def kernel(v_e, v_score, table):
    raise NotImplementedError("write your pallas kernel here")


# speedup vs baseline: 1.4304x; 1.4304x over previous
import functools

import jax
import jax.numpy as jnp
from jax import lax
from jax.experimental import pallas as pl
from jax.experimental.pallas import tpu as pltpu
from jax.experimental.pallas import tpu_sc as plsc


def _make_sc_kernel(n_total: int, d: int, c_chunk: int):
    info = plsc.get_sparse_core_info()
    nc, ns = info.num_cores, info.num_subcores
    nw = nc * ns
    assert n_total % nw == 0
    b_per_w = n_total // nw
    assert b_per_w % c_chunk == 0
    n_chunks = b_per_w // c_chunk
    mesh = plsc.VectorSubcoreMesh(core_axis_name="c", subcore_axis_name="s")

    @functools.partial(
        pl.kernel,
        mesh=mesh,
        out_type=jax.ShapeDtypeStruct((n_total, d), jnp.float32),
        compiler_params=pltpu.CompilerParams(use_tc_tiling_on_sc=False),
        scratch_types=[
            pltpu.VMEM((c_chunk,), jnp.int32),
            pltpu.VMEM((c_chunk,), jnp.float32),
            pltpu.VMEM((c_chunk, d), jnp.float32),
            pltpu.SemaphoreType.DMA,
        ],
    )
    def sc_kernel(idx_hbm, score_hbm, table_hbm, out_hbm,
                  idx_v, score_v, rows_v, sem):
        wid = lax.axis_index("s") * nc + lax.axis_index("c")
        base = wid * b_per_w

        def chunk_body(g, carry):
            off = base + g * c_chunk
            pltpu.sync_copy(idx_hbm.at[pl.ds(off, c_chunk)], idx_v)
            pltpu.sync_copy(score_hbm.at[pl.ds(off, c_chunk)], score_v)
            pltpu.async_copy(table_hbm.at[idx_v], rows_v, sem).wait()

            def row16_body(r, c):
                i = r * 16
                s_vec = score_v[pl.ds(i, 16)]
                for j in range(16):
                    s = s_vec[j]
                    rows_v[i + j, pl.ds(0, 16)] = rows_v[i + j, pl.ds(0, 16)] * s
                    rows_v[i + j, pl.ds(16, 16)] = rows_v[i + j, pl.ds(16, 16)] * s
                return c

            lax.fori_loop(0, c_chunk // 16, row16_body, 0)
            pltpu.sync_copy(rows_v, out_hbm.at[pl.ds(off, c_chunk)])
            return carry

        lax.fori_loop(0, n_chunks, chunk_body, 0)

    return sc_kernel


def kernel(v_e, v_score, table):
    b, h = v_e.shape
    v, d = table.shape
    n = b * h
    idx = v_e.reshape(n).astype(jnp.int32)
    score = v_score.reshape(n).astype(jnp.float32)
    out = _make_sc_kernel(n, d, c_chunk=3200)(idx, score, table)
    return out.reshape(b, h, d)
